# Initial kernel scaffold; baseline (speedup 1.0000x reference)
import jax
import jax.numpy as jnp
from jax.experimental import pallas as pl


def kernel(features, W1, b1, W2, b2, memory_bank):
    raise NotImplementedError("write your pallas kernel here")


# speedup vs baseline: 2.0384x; 2.0384x over previous
import jax
import jax.numpy as jnp
from jax.experimental import pallas as pl
from jax.experimental.pallas import tpu as pltpu

_BM = 1024


def _body(xT_ref, w1t_ref, b1_ref, w2t_ref, b2_ref, bank_ref,
          out_ref, pT_ref, acc_ref):
    i = pl.program_id(0)
    nm = pl.num_programs(0)

    @pl.when(i == 0)
    def _init():
        hT = jnp.maximum(
            jnp.dot(w1t_ref[...], xT_ref[...],
                    preferred_element_type=jnp.float32) + b1_ref[...], 0.0)
        pT_ref[...] = jnp.dot(w2t_ref[...], hT,
                              preferred_element_type=jnp.float32) + b2_ref[...]
        acc_ref[...] = jnp.full_like(acc_ref[...], jnp.inf)

    bank = bank_ref[...]
    pT = pT_ref[...]
    cross = jnp.dot(bank, pT, preferred_element_type=jnp.float32)
    m_sq = jnp.sum(bank * bank, axis=1, keepdims=True)
    t = m_sq - 2.0 * cross
    acc_ref[...] = jnp.minimum(acc_ref[...], jnp.min(t, axis=0, keepdims=True))

    @pl.when(i == nm - 1)
    def _fin():
        p_sq = jnp.sum(pT_ref[...] * pT_ref[...], axis=0, keepdims=True)
        out_ref[...] = jnp.sqrt(jnp.maximum(acc_ref[...] + p_sq, 1e-12))


def kernel(features, W1, b1, W2, b2, memory_bank):
    B, L, C = features.shape
    N = B * L
    M, d2 = memory_bank.shape
    d1 = W1.shape[1]

    xT = features.reshape(N, C).T
    w1t = W1.T
    w2t = W2.T
    b1c = b1[:, None]
    b2c = b2[:, None]

    mpad = ((M + _BM - 1) // _BM) * _BM
    bank = jnp.pad(memory_bank, ((0, mpad - M), (0, 0)), constant_values=1e6)

    grid = (mpad // _BM,)
    out = pl.pallas_call(
        _body,
        grid=grid,
        in_specs=[
            pl.BlockSpec((C, N), lambda i: (0, 0)),
            pl.BlockSpec((d1, C), lambda i: (0, 0)),
            pl.BlockSpec((d1, 1), lambda i: (0, 0)),
            pl.BlockSpec((d2, d1), lambda i: (0, 0)),
            pl.BlockSpec((d2, 1), lambda i: (0, 0)),
            pl.BlockSpec((_BM, d2), lambda i: (i, 0)),
        ],
        out_specs=pl.BlockSpec((1, N), lambda i: (0, 0)),
        out_shape=jax.ShapeDtypeStruct((1, N), jnp.float32),
        scratch_shapes=[
            pltpu.VMEM((d2, N), jnp.float32),
            pltpu.VMEM((1, N), jnp.float32),
        ],
    )(xT, w1t, b1c, w2t, b2c, bank)
    return out.reshape(B, L)
